# baseline (speedup 1.0000x reference)
import functools

import jax
import jax.numpy as jnp
from jax import lax
from jax.experimental import pallas as pl
from jax.experimental.pallas import tpu as pltpu
from jax.experimental.pallas import tpu_sc as plsc

B = 16384
L = 50
D = 64
NC = 2
NS = 16
NW = NC * NS
SEQ_PER_W = B // NW
SEQ_PER_CHUNK = 16
NCHUNKS = SEQ_PER_W // SEQ_PER_CHUNK
LANES = 16
DGRP = D // LANES


def _emb_body(x_hbm, table_hbm, pe_hbm, out_hbm,
              idx0, idx1, rows0, rows1, pe_v,
              gsem0, gsem1, wsem0, wsem1):
    wid = lax.axis_index("s") * NC + lax.axis_index("c")
    seq_base = wid * SEQ_PER_W

    idx = (idx0, idx1)
    rows = (rows0, rows1)
    gsem = (gsem0, gsem1)
    wsem = (wsem0, wsem1)

    pltpu.sync_copy(pe_hbm.at[pl.ds(0, L)], pe_v)

    def load_idx(g, b):
        s0 = seq_base + g * SEQ_PER_CHUNK
        pltpu.sync_copy(x_hbm.at[pl.ds(s0, SEQ_PER_CHUNK)], idx[b])

    def fire_gather(b):
        for s in range(SEQ_PER_CHUNK):
            pltpu.async_copy(table_hbm.at[idx[b].at[s]], rows[b].at[s],
                             gsem[b])

    def wait_gather(b):
        for s in range(SEQ_PER_CHUNK):
            pltpu.make_async_copy(table_hbm.at[idx[b].at[s]], rows[b].at[s],
                                  gsem[b]).wait()

    def add_pe(b):
        rv = rows[b]

        def pos_body(l, c2):
            for c in range(DGRP):
                pe_vec = pe_v[l, pl.ds(c * LANES, LANES)]
                for s in range(SEQ_PER_CHUNK):
                    rv[s, l, pl.ds(c * LANES, LANES)] = (
                        rv[s, l, pl.ds(c * LANES, LANES)] + pe_vec
                    )
            return c2

        lax.fori_loop(0, L, pos_body, 0, unroll=False)

    def fire_wb(g, b):
        s0 = seq_base + g * SEQ_PER_CHUNK
        pltpu.async_copy(rows[b], out_hbm.at[pl.ds(s0, SEQ_PER_CHUNK)],
                         wsem[b])

    def wait_wb(g, b):
        s0 = seq_base + g * SEQ_PER_CHUNK
        pltpu.make_async_copy(rows[b],
                              out_hbm.at[pl.ds(s0, SEQ_PER_CHUNK)],
                              wsem[b]).wait()

    load_idx(0, 0)
    fire_gather(0)
    load_idx(1, 1)
    fire_gather(1)
    wait_gather(0)
    add_pe(0)
    fire_wb(0, 0)

    def pair_body(k, carry):
        g = 1 + 2 * k
        load_idx(g + 1, 0)
        wait_wb(g - 1, 0)
        fire_gather(0)
        wait_gather(1)
        add_pe(1)
        fire_wb(g, 1)
        load_idx(g + 2, 1)
        wait_wb(g, 1)
        fire_gather(1)
        wait_gather(0)
        add_pe(0)
        fire_wb(g + 1, 0)
        return carry

    lax.fori_loop(0, (NCHUNKS - 2) // 2 - 1, pair_body, 0, unroll=False)

    g = NCHUNKS - 3
    load_idx(g + 1, 0)
    wait_wb(g - 1, 0)
    fire_gather(0)
    wait_gather(1)
    add_pe(1)
    fire_wb(g, 1)
    load_idx(g + 2, 1)
    wait_wb(g, 1)
    fire_gather(1)
    wait_gather(0)
    add_pe(0)
    fire_wb(g + 1, 0)

    wait_gather(1)
    add_pe(1)
    fire_wb(NCHUNKS - 1, 1)
    wait_wb(NCHUNKS - 2, 0)
    wait_wb(NCHUNKS - 1, 1)


@jax.jit
def _emb_call(x, table, pe):
    mesh = plsc.VectorSubcoreMesh(core_axis_name="c", subcore_axis_name="s")
    run = pl.kernel(
        _emb_body,
        out_type=jax.ShapeDtypeStruct((B, L, D), jnp.float32),
        mesh=mesh,
        scratch_types=[
            pltpu.VMEM((SEQ_PER_CHUNK, L), jnp.int32),
            pltpu.VMEM((SEQ_PER_CHUNK, L), jnp.int32),
            pltpu.VMEM((SEQ_PER_CHUNK, L, D), jnp.float32),
            pltpu.VMEM((SEQ_PER_CHUNK, L, D), jnp.float32),
            pltpu.VMEM((L, D), jnp.float32),
            pltpu.SemaphoreType.DMA,
            pltpu.SemaphoreType.DMA,
            pltpu.SemaphoreType.DMA,
            pltpu.SemaphoreType.DMA,
        ],
        compiler_params=pltpu.CompilerParams(use_tc_tiling_on_sc=False),
    )
    return run(x, table, pe)


def kernel(x, table, pe):
    return _emb_call(x, table, pe)


# speedup vs baseline: 1.0857x; 1.0857x over previous
import functools

import jax
import jax.numpy as jnp
from jax import lax
from jax.experimental import pallas as pl
from jax.experimental import layout as jlayout
from jax.experimental.pallas import tpu as pltpu
from jax.experimental.pallas import tpu_sc as plsc

B = 16384
L = 50
D = 64
DP = 128
NC = 2
NS = 16
NW = NC * NS
SEQ_PER_W = B // NW
SEQ_PER_BLOCK = 8
SEQ_PER_CHUNK = 4
NBLOCKS = SEQ_PER_W // SEQ_PER_BLOCK
NCHUNKS = SEQ_PER_W // SEQ_PER_CHUNK
LANES = 16
DGRP = D // LANES
PE_ROWS = 56


def _emb_body(x_hbm, table_hbm, pe_hbm, out_hbm,
              idx0, idx1, rows0, rows1, st0, st1, pe_v,
              gsem0, gsem1, wsem0, wsem1):
    wid = lax.axis_index("s") * NC + lax.axis_index("c")
    seq_base = wid * SEQ_PER_W

    idx = (idx0, idx1)
    rows = (rows0, rows1)
    stg = (st0, st1)
    gsem = (gsem0, gsem1)
    wsem = (wsem0, wsem1)

    pltpu.sync_copy(pe_hbm.at[pl.ds(0, PE_ROWS)], pe_v)

    def load_block(h, ib):
        s0 = seq_base + h * SEQ_PER_BLOCK
        pltpu.sync_copy(x_hbm.at[pl.ds(s0, SEQ_PER_BLOCK)], idx[ib])

    def fire_gather(rb, ib, half):
        for s in range(SEQ_PER_CHUNK):
            pltpu.async_copy(
                table_hbm.at[idx[ib].at[half * SEQ_PER_CHUNK + s]],
                rows[rb].at[s], gsem[rb])

    def wait_gather(rb, ib, half):
        for s in range(SEQ_PER_CHUNK):
            pltpu.make_async_copy(
                table_hbm.at[idx[ib].at[half * SEQ_PER_CHUNK + s]],
                rows[rb].at[s], gsem[rb]).wait()

    def add_pe(b):
        rv = rows[b]
        sv = stg[b]

        def pos_body(l, c2):
            for c in range(DGRP):
                pe_vec = pe_v[l, pl.ds(c * LANES, LANES)]
                for s in range(SEQ_PER_CHUNK):
                    sv[s, l, pl.ds(c * LANES, LANES)] = (
                        rv[s, l, pl.ds(c * LANES, LANES)] + pe_vec
                    )
            return c2

        lax.fori_loop(0, L, pos_body, 0, unroll=False)

    def fire_wb(g, b):
        s0 = seq_base + g * SEQ_PER_CHUNK
        pltpu.async_copy(stg[b], out_hbm.at[pl.ds(s0, SEQ_PER_CHUNK)],
                         wsem[b])

    def wait_wb(g, b):
        s0 = seq_base + g * SEQ_PER_CHUNK
        pltpu.make_async_copy(stg[b],
                              out_hbm.at[pl.ds(s0, SEQ_PER_CHUNK)],
                              wsem[b]).wait()


    load_block(0, 0)
    fire_gather(0, 0, 0)
    fire_gather(1, 0, 1)
    wait_gather(0, 0, 0)
    add_pe(0)
    fire_wb(0, 0)

    def body(j, carry):
        g = 4 * j + 1
        load_block(2 * j + 1, 1)
        fire_gather(0, 1, 0)
        wait_gather(1, 0, 1)
        wait_wb(g - 2, 1)
        add_pe(1)
        fire_wb(g, 1)
        fire_gather(1, 1, 1)
        wait_gather(0, 1, 0)
        wait_wb(g - 1, 0)
        add_pe(0)
        fire_wb(g + 1, 0)
        load_block(2 * j + 2, 0)
        fire_gather(0, 0, 0)
        wait_gather(1, 1, 1)
        wait_wb(g, 1)
        add_pe(1)
        fire_wb(g + 2, 1)
        fire_gather(1, 0, 1)
        wait_gather(0, 0, 0)
        wait_wb(g + 1, 0)
        add_pe(0)
        fire_wb(g + 3, 0)
        return carry

    def body0():
        load_block(1, 1)
        fire_gather(0, 1, 0)
        wait_gather(1, 0, 1)
        add_pe(1)
        fire_wb(1, 1)
        fire_gather(1, 1, 1)
        wait_gather(0, 1, 0)
        wait_wb(0, 0)
        add_pe(0)
        fire_wb(2, 0)
        load_block(2, 0)
        fire_gather(0, 0, 0)
        wait_gather(1, 1, 1)
        wait_wb(1, 1)
        add_pe(1)
        fire_wb(3, 1)
        fire_gather(1, 0, 1)
        wait_gather(0, 0, 0)
        wait_wb(2, 0)
        add_pe(0)
        fire_wb(4, 0)

    body0()
    lax.fori_loop(1, (NCHUNKS - 4) // 4, body, 0, unroll=False)

    load_block(NBLOCKS - 1, 1)
    fire_gather(0, 1, 0)
    wait_gather(1, 0, 1)
    wait_wb(123, 1)
    add_pe(1)
    fire_wb(125, 1)
    fire_gather(1, 1, 1)
    wait_gather(0, 1, 0)
    wait_wb(124, 0)
    add_pe(0)
    fire_wb(126, 0)
    wait_gather(1, 1, 1)
    wait_wb(125, 1)
    add_pe(1)
    fire_wb(127, 1)
    wait_wb(126, 0)
    wait_wb(127, 1)


def _emb_full(x, table, pe):
    tpad = jnp.pad(table, ((0, 0), (0, DP - D)))
    mesh = plsc.VectorSubcoreMesh(core_axis_name="c", subcore_axis_name="s")
    run = pl.kernel(
        _emb_body,
        out_type=jax.ShapeDtypeStruct((B, L, D), jnp.float32),
        mesh=mesh,
        scratch_types=[
            pltpu.VMEM((SEQ_PER_BLOCK, L), jnp.int32),
            pltpu.VMEM((SEQ_PER_BLOCK, L), jnp.int32),
            pltpu.VMEM((SEQ_PER_CHUNK, L, DP), jnp.float32),
            pltpu.VMEM((SEQ_PER_CHUNK, L, DP), jnp.float32),
            pltpu.VMEM((SEQ_PER_CHUNK, L, D), jnp.float32),
            pltpu.VMEM((SEQ_PER_CHUNK, L, D), jnp.float32),
            pltpu.VMEM((PE_ROWS, D), jnp.float32),
            pltpu.SemaphoreType.DMA,
            pltpu.SemaphoreType.DMA,
            pltpu.SemaphoreType.DMA,
            pltpu.SemaphoreType.DMA,
        ],
    )
    return run(x, tpad, pe)


def kernel(x, table, pe):
    return _emb_full(x, table, pe)


# speedup vs baseline: 1.2244x; 1.1277x over previous
import functools

import jax
import jax.numpy as jnp
from jax import lax
from jax.experimental import pallas as pl
from jax.experimental.pallas import tpu as pltpu
from jax.experimental.pallas import tpu_sc as plsc

B = 16384
L = 50
D = 64
DP = 128
NC = 2
NS = 16
NW = NC * NS
SEQ_PER_W = B // NW
SEQ_PER_BLOCK = 8
SEQ_PER_CHUNK = 4
NBLOCKS = SEQ_PER_W // SEQ_PER_BLOCK
NCHUNKS = SEQ_PER_W // SEQ_PER_CHUNK
LANES = 16
DGRP = D // LANES
PE_ROWS = 56


def _emb_body(x_hbm, table_hbm, pe_hbm, out_hbm,
              idx0, idx1, rows0, rows1, st0, st1, pe_v,
              gsem0, gsem1, wsem0, wsem1):
    wid = lax.axis_index("s") * NC + lax.axis_index("c")
    seq_base = wid * SEQ_PER_W

    idx = (idx0, idx1)
    rows = (rows0, rows1)
    stg = (st0, st1)
    gsem = (gsem0, gsem1)
    wsem = (wsem0, wsem1)

    pltpu.sync_copy(pe_hbm.at[pl.ds(0, PE_ROWS)], pe_v)

    def load_block(h, ib):
        s0 = seq_base + h * SEQ_PER_BLOCK
        pltpu.sync_copy(x_hbm.at[pl.ds(s0, SEQ_PER_BLOCK)], idx[ib])

    def fire_gather(rb, ib, half):
        for s in range(SEQ_PER_CHUNK):
            pltpu.async_copy(
                table_hbm.at[idx[ib].at[half * SEQ_PER_CHUNK + s]],
                rows[rb].at[s], gsem[rb])

    def wait_gather(rb, ib, half):
        for s in range(SEQ_PER_CHUNK):
            pltpu.make_async_copy(
                table_hbm.at[idx[ib].at[half * SEQ_PER_CHUNK + s]],
                rows[rb].at[s], gsem[rb]).wait()

    def add_pe(b):
        rv = rows[b]
        sv = stg[b]

        def pos_body(l, c2):
            for c in range(DGRP):
                pe_vec = pe_v[l, pl.ds(c * LANES, LANES)]
                for s in range(SEQ_PER_CHUNK):
                    sv[s, l, pl.ds(c * LANES, LANES)] = (
                        rv[s, l, pl.ds(c * LANES, LANES)] + pe_vec
                    )
            return c2

        lax.fori_loop(0, L, pos_body, 0, unroll=False)

    def fire_wb(g, b):
        s0 = seq_base + g * SEQ_PER_CHUNK
        pltpu.async_copy(stg[b], out_hbm.at[pl.ds(s0, SEQ_PER_CHUNK)],
                         wsem[b])

    def wait_wb(g, b):
        s0 = seq_base + g * SEQ_PER_CHUNK
        pltpu.make_async_copy(stg[b],
                              out_hbm.at[pl.ds(s0, SEQ_PER_CHUNK)],
                              wsem[b]).wait()


    load_block(0, 0)
    fire_gather(0, 0, 0)
    fire_gather(1, 0, 1)
    wait_gather(0, 0, 0)
    add_pe(0)
    fire_wb(0, 0)

    def body(j, carry):
        g = 4 * j + 1
        load_block(2 * j + 1, 1)
        fire_gather(0, 1, 0)
        wait_gather(1, 0, 1)
        wait_wb(g - 2, 1)
        add_pe(1)
        fire_wb(g, 1)
        fire_gather(1, 1, 1)
        wait_gather(0, 1, 0)
        wait_wb(g - 1, 0)
        add_pe(0)
        fire_wb(g + 1, 0)
        load_block(2 * j + 2, 0)
        fire_gather(0, 0, 0)
        wait_gather(1, 1, 1)
        wait_wb(g, 1)
        add_pe(1)
        fire_wb(g + 2, 1)
        fire_gather(1, 0, 1)
        wait_gather(0, 0, 0)
        wait_wb(g + 1, 0)
        add_pe(0)
        fire_wb(g + 3, 0)
        return carry

    def body0():
        load_block(1, 1)
        fire_gather(0, 1, 0)
        wait_gather(1, 0, 1)
        add_pe(1)
        fire_wb(1, 1)
        fire_gather(1, 1, 1)
        wait_gather(0, 1, 0)
        wait_wb(0, 0)
        add_pe(0)
        fire_wb(2, 0)
        load_block(2, 0)
        fire_gather(0, 0, 0)
        wait_gather(1, 1, 1)
        wait_wb(1, 1)
        add_pe(1)
        fire_wb(3, 1)
        fire_gather(1, 0, 1)
        wait_gather(0, 0, 0)
        wait_wb(2, 0)
        add_pe(0)
        fire_wb(4, 0)

    body0()
    lax.fori_loop(1, (NCHUNKS - 4) // 4, body, 0, unroll=False)

    load_block(NBLOCKS - 1, 1)
    fire_gather(0, 1, 0)
    wait_gather(1, 0, 1)
    wait_wb(123, 1)
    add_pe(1)
    fire_wb(125, 1)
    fire_gather(1, 1, 1)
    wait_gather(0, 1, 0)
    wait_wb(124, 0)
    add_pe(0)
    fire_wb(126, 0)
    wait_gather(1, 1, 1)
    wait_wb(125, 1)
    add_pe(1)
    fire_wb(127, 1)
    wait_wb(126, 0)
    wait_wb(127, 1)


@jax.jit
def _emb_full(x, table, pe):
    tpad = jnp.pad(table, ((0, 0), (0, DP - D)))
    mesh = plsc.VectorSubcoreMesh(core_axis_name="c", subcore_axis_name="s")
    run = pl.kernel(
        _emb_body,
        out_type=jax.ShapeDtypeStruct((B, L, D), jnp.float32),
        mesh=mesh,
        scratch_types=[
            pltpu.VMEM((SEQ_PER_BLOCK, L), jnp.int32),
            pltpu.VMEM((SEQ_PER_BLOCK, L), jnp.int32),
            pltpu.VMEM((SEQ_PER_CHUNK, L, DP), jnp.float32),
            pltpu.VMEM((SEQ_PER_CHUNK, L, DP), jnp.float32),
            pltpu.VMEM((SEQ_PER_CHUNK, L, D), jnp.float32),
            pltpu.VMEM((SEQ_PER_CHUNK, L, D), jnp.float32),
            pltpu.VMEM((PE_ROWS, D), jnp.float32),
            pltpu.SemaphoreType.DMA,
            pltpu.SemaphoreType.DMA,
            pltpu.SemaphoreType.DMA,
            pltpu.SemaphoreType.DMA,
        ],
    )
    out = run(x, tpad, pe)
    return lax.optimization_barrier(out)


def kernel(x, table, pe):
    return _emb_full(x, table, pe)
